# baseline (speedup 1.0000x reference)
import functools

import jax
import jax.numpy as jnp
from jax import lax
from jax.experimental import pallas as pl
from jax.experimental.pallas import tpu as pltpu
from jax.experimental.pallas import tpu_sc as plsc

NY, NX, C = 162, 162, 128
NB = 4
S = NY * NX
SP = 26624
TOT = NB * SP
P = 40000

NTILES = 32
RANGE = TOT // NTILES
CHUNK = 2512
PPAD = 16 * CHUNK
NGRP = PPAD // 16
LCAP = RANGE + 128
KCH = 3

_mesh = plsc.VectorSubcoreMesh(core_axis_name="c", subcore_axis_name="s")


@functools.partial(
    pl.kernel,
    out_type=[
        jax.ShapeDtypeStruct((TOT + 128, C), jnp.float32),
        jax.ShapeDtypeStruct((TOT,), jnp.int32),
    ],
    mesh=_mesh,
    scratch_types=[
        pltpu.VMEM((PPAD,), jnp.int32),
        pltpu.VMEM((RANGE,), jnp.int32),
        pltpu.VMEM((4 * CHUNK,), jnp.int32),
        pltpu.VMEM((KCH * 128, C), jnp.float32),
        pltpu.VMEM((LCAP,), jnp.int32),
        pltpu.VMEM((LCAP,), jnp.int32),
        pltpu.VMEM((LCAP // 128, 128), jnp.int32),
        pltpu.VMEM_SHARED((PPAD,), jnp.int32),
        pltpu.SemaphoreType.DMA,
        pltpu.SemaphoreType.DMA,
    ],
    compiler_params=pltpu.CompilerParams(needs_layout_passes=False),
)
def _k1(coors_hbm, vf_hbm, staging_hbm, winner_hbm,
        g2_v, winner_v, coorsw_v, rows_v, wl_v, sl_v, sl2_v, g2_sp,
        gsem, ssem):
    sid = lax.axis_index("s")
    wid = lax.axis_index("c") * 16 + sid
    lo = wid * RANGE
    lane = jnp.arange(16, dtype=jnp.int32)
    neg1 = jnp.full((16,), -1, jnp.int32)

    cb = sid * CHUNK

    @pl.when(sid < 15)
    def _():
        pltpu.sync_copy(coors_hbm.at[pl.ds(cb * 4, 4 * CHUNK)], coorsw_v)

    @pl.when(sid == 15)
    def _():
        n_tail = 4 * (P - 15 * CHUNK)
        pltpu.sync_copy(coors_hbm.at[pl.ds(cb * 4, n_tail)],
                        coorsw_v.at[pl.ds(0, n_tail)])

    def _ga(i, _):
        off = i * 64 + lane * 4
        b = plsc.load_gather(coorsw_v, [off]) & 3
        y = plsc.load_gather(coorsw_v, [off + 2])
        x = plsc.load_gather(coorsw_v, [off + 3])
        g2 = b * SP + y * NX + x
        pmask = (cb + i * 16 + lane) < P
        g2_v[pl.ds(i * 16, 16)] = jnp.where(pmask, g2, TOT)
        return 0
    lax.fori_loop(0, CHUNK // 16, _ga, 0)
    pltpu.sync_copy(g2_v.at[pl.ds(0, CHUNK)], g2_sp.at[pl.ds(cb, CHUNK)])
    plsc.subcore_barrier()

    def _init(i, _):
        winner_v[pl.ds(i * 16, 16)] = neg1
        return 0
    lax.fori_loop(0, RANGE // 16, _init, 0)

    pltpu.sync_copy(g2_sp, g2_v)

    def _scan(i4, _):
        for u in range(8):
            i = i4 * 8 + u
            il = g2_v[pl.ds(i * 16, 16)] - lo
            m = (il >= 0) & (il < RANGE)
            ilc = jnp.where(m, il, 0)
            p_vec = i * 16 + lane
            plsc.store_scatter(winner_v, [ilc], p_vec, mask=m)
        return 0
    lax.fori_loop(0, NGRP // 8, _scan, 0)

    pltpu.sync_copy(winner_v, winner_hbm.at[pl.ds(lo, RANGE)])

    def _pref(i, _):
        pos = i * 16 + lane
        wl_v[pl.ds(i * 16, 16)] = (pos + wid * 128) & 16383
        sl_v[pl.ds(i * 16, 16)] = TOT + ((pos + wid * 4) & 127)
        return 0
    lax.fori_loop(0, LCAP // 16, _pref, 0)

    def _cmp(j, acc):
        w16 = winner_v[pl.ds(j * 16, 16)]
        valid = w16 >= 0
        plsc.store_compressed(wl_v.at[pl.ds(acc, 16)], w16, mask=valid)
        slot = lo + j * 16 + lane
        plsc.store_compressed(sl_v.at[pl.ds(acc, 16)], slot, mask=valid)
        return acc + jnp.sum(valid.astype(jnp.int32))
    nv = lax.fori_loop(0, RANGE // 16, _cmp, jnp.int32(0))

    def _c2o(kk, _):
        def _c2i(j, _):
            sl2_v[kk, pl.ds(j * 16, 16)] = sl_v[pl.ds(kk * 128 + j * 16, 16)]
            return 0
        lax.fori_loop(0, 8, _c2i, 0)
        return 0
    lax.fori_loop(0, LCAP // 128, _c2o, 0)

    nsc = (nv + (KCH * 128 - 1)) // (KCH * 128)

    def _sc(t, _):
        base = t * (KCH * 128)
        hs = [pltpu.async_copy(
                  vf_hbm.at[wl_v.at[pl.ds(base + q * 128, 128)]],
                  rows_v.at[pl.ds(q * 128, 128), :], gsem)
              for q in range(KCH)]
        for h in hs:
            h.wait()
        hs2 = [pltpu.async_copy(
                   rows_v.at[pl.ds(q * 128, 128), :],
                   staging_hbm.at[sl2_v.at[t * KCH + q]], ssem)
               for q in range(KCH)]
        for h in hs2:
            h.wait()
        return 0
    lax.fori_loop(0, nsc, _sc, 0)


SCH = 13312


def _k2_body(wref, sref, oref):
    x = sref[...]
    wm = wref[0, 0]
    xt = x.T
    oref[0] = jnp.where((wm >= 0)[None, :], xt, 0.0)


_k2 = pl.pallas_call(
    _k2_body,
    grid=(NB, SP // SCH),
    in_specs=[
        pl.BlockSpec((1, 1, SCH), lambda b, s: ((SP // SCH) * b + s, 0, 0)),
        pl.BlockSpec((SCH, C), lambda b, s: ((SP // SCH) * b + s, 0)),
    ],
    out_specs=pl.BlockSpec((1, C, SCH), lambda b, s: (b, 0, s)),
    out_shape=jax.ShapeDtypeStruct((NB, C, S), jnp.float32),
    compiler_params=pltpu.CompilerParams(
        dimension_semantics=("parallel", "parallel")),
)


def kernel(voxel_features, coors, batch_size):
    del batch_size
    coors_flat = coors.reshape(-1)
    staging, winner = _k1(coors_flat, voxel_features)
    out = _k2(winner.reshape(TOT // SCH, 1, SCH), staging)
    return out.reshape(NB, C, NY, NX)


# speedup vs baseline: 1.0428x; 1.0428x over previous
import functools

import jax
import jax.numpy as jnp
from jax import lax
from jax.experimental import pallas as pl
from jax.experimental.pallas import tpu as pltpu
from jax.experimental.pallas import tpu_sc as plsc

NY, NX, C = 162, 162, 128
NB = 4
S = NY * NX
SP = 26624
TOT = NB * SP
P = 40000

NTILES = 32
RANGE = TOT // NTILES
CHUNK = 2512
PPAD = 16 * CHUNK
NGRP = PPAD // 16
LCAP = RANGE + 128
KCH = 3

_mesh = plsc.VectorSubcoreMesh(core_axis_name="c", subcore_axis_name="s")


@functools.partial(
    pl.kernel,
    out_type=[
        jax.ShapeDtypeStruct((TOT + 128, C), jnp.float32),
        jax.ShapeDtypeStruct((TOT,), jnp.int32),
    ],
    mesh=_mesh,
    scratch_types=[
        pltpu.VMEM((PPAD // 4,), jnp.int32),
        pltpu.VMEM((RANGE,), jnp.int32),
        pltpu.VMEM((TOT // 8,), jnp.int32),
        pltpu.VMEM((RANGE,), jnp.int32),
        pltpu.VMEM((4 * CHUNK,), jnp.int32),
        pltpu.VMEM((KCH * 128, C), jnp.float32),
        pltpu.VMEM((LCAP,), jnp.int32),
        pltpu.VMEM((LCAP,), jnp.int32),
        pltpu.VMEM((LCAP // 128, 128), jnp.int32),
        pltpu.VMEM_SHARED((PPAD,), jnp.int32),
        pltpu.VMEM_SHARED((16, TOT // 8), jnp.int32),
        pltpu.SemaphoreType.DMA,
        pltpu.SemaphoreType.DMA,
    ],
    compiler_params=pltpu.CompilerParams(needs_layout_passes=False),
)
def _k1(coors_hbm, vf_hbm, staging_hbm, winner_hbm,
        g2_v, winner_v, part_v, tmp_v, coorsw_v, rows_v, wl_v, sl_v,
        sl2_v, g2_sp, part_sp, gsem, ssem):
    sid = lax.axis_index("s")
    wid = lax.axis_index("c") * 16 + sid
    lo = wid * RANGE
    lane = jnp.arange(16, dtype=jnp.int32)
    neg1 = jnp.full((16,), -1, jnp.int32)

    cb = sid * CHUNK

    @pl.when(sid < 15)
    def _():
        pltpu.sync_copy(coors_hbm.at[pl.ds(cb * 4, 4 * CHUNK)], coorsw_v)

    @pl.when(sid == 15)
    def _():
        n_tail = 4 * (P - 15 * CHUNK)
        pltpu.sync_copy(coors_hbm.at[pl.ds(cb * 4, n_tail)],
                        coorsw_v.at[pl.ds(0, n_tail)])

    def _ga(i, _):
        off = i * 64 + lane * 4
        b = plsc.load_gather(coorsw_v, [off]) & 3
        y = plsc.load_gather(coorsw_v, [off + 2])
        x = plsc.load_gather(coorsw_v, [off + 3])
        g2 = b * SP + y * NX + x
        pmask = (cb + i * 16 + lane) < P
        g2_v[pl.ds(i * 16, 16)] = jnp.where(pmask, g2, TOT)
        return 0
    lax.fori_loop(0, CHUNK // 16, _ga, 0)
    pltpu.sync_copy(g2_v.at[pl.ds(0, CHUNK)], g2_sp.at[pl.ds(cb, CHUNK)])
    plsc.subcore_barrier()

    CL = TOT // 8
    QP = PPAD // 4
    cl = sid // 4
    qt = sid % 4
    clo = lax.axis_index("c") * (TOT // 2) + cl * CL

    def _init(i, _):
        part_v[pl.ds(i * 16, 16)] = neg1
        return 0
    lax.fori_loop(0, CL // 16, _init, 0)

    pltpu.sync_copy(g2_sp.at[pl.ds(qt * QP, QP)], g2_v)

    def _scan(i4, _):
        for u in range(8):
            i = i4 * 8 + u
            il = g2_v[pl.ds(i * 16, 16)] - clo
            m = (il >= 0) & (il < CL)
            ilc = jnp.where(m, il, 0)
            p_vec = qt * QP + i * 16 + lane
            plsc.store_scatter(part_v, [ilc], p_vec, mask=m)
        return 0
    lax.fori_loop(0, QP // 16 // 8, _scan, 0)

    pltpu.sync_copy(part_v, part_sp.at[sid])
    plsc.subcore_barrier()

    sub = qt * RANGE
    pltpu.sync_copy(part_sp.at[cl * 4, pl.ds(sub, RANGE)], winner_v)
    for q2 in range(1, 4):
        pltpu.sync_copy(part_sp.at[cl * 4 + q2, pl.ds(sub, RANGE)], tmp_v)

        def _mx(i, _):
            winner_v[pl.ds(i * 16, 16)] = jnp.maximum(
                winner_v[pl.ds(i * 16, 16)], tmp_v[pl.ds(i * 16, 16)])
            return 0
        lax.fori_loop(0, RANGE // 16, _mx, 0)

    pltpu.sync_copy(winner_v, winner_hbm.at[pl.ds(lo, RANGE)])

    def _pref(i, _):
        pos = i * 16 + lane
        wl_v[pl.ds(i * 16, 16)] = (pos + wid * 128) & 16383
        sl_v[pl.ds(i * 16, 16)] = TOT + ((pos + wid * 4) & 127)
        return 0
    lax.fori_loop(0, LCAP // 16, _pref, 0)

    def _cmp(j, acc):
        w16 = winner_v[pl.ds(j * 16, 16)]
        valid = w16 >= 0
        plsc.store_compressed(wl_v.at[pl.ds(acc, 16)], w16, mask=valid)
        slot = lo + j * 16 + lane
        plsc.store_compressed(sl_v.at[pl.ds(acc, 16)], slot, mask=valid)
        return acc + jnp.sum(valid.astype(jnp.int32))
    nv = lax.fori_loop(0, RANGE // 16, _cmp, jnp.int32(0))

    def _c2o(kk, _):
        def _c2i(j, _):
            sl2_v[kk, pl.ds(j * 16, 16)] = sl_v[pl.ds(kk * 128 + j * 16, 16)]
            return 0
        lax.fori_loop(0, 8, _c2i, 0)
        return 0
    lax.fori_loop(0, LCAP // 128, _c2o, 0)

    nsc = (nv + (KCH * 128 - 1)) // (KCH * 128)

    def _sc(t, _):
        base = t * (KCH * 128)
        hs = [pltpu.async_copy(
                  vf_hbm.at[wl_v.at[pl.ds(base + q * 128, 128)]],
                  rows_v.at[pl.ds(q * 128, 128), :], gsem)
              for q in range(KCH)]
        for h in hs:
            h.wait()
        hs2 = [pltpu.async_copy(
                   rows_v.at[pl.ds(q * 128, 128), :],
                   staging_hbm.at[sl2_v.at[t * KCH + q]], ssem)
               for q in range(KCH)]
        for h in hs2:
            h.wait()
        return 0
    lax.fori_loop(0, nsc, _sc, 0)


SCH = 13312


def _k2_body(wref, sref, oref):
    x = sref[...]
    wm = wref[0, 0]
    xt = x.T
    oref[0] = jnp.where((wm >= 0)[None, :], xt, 0.0)


_k2 = pl.pallas_call(
    _k2_body,
    grid=(NB, SP // SCH),
    in_specs=[
        pl.BlockSpec((1, 1, SCH), lambda b, s: ((SP // SCH) * b + s, 0, 0)),
        pl.BlockSpec((SCH, C), lambda b, s: ((SP // SCH) * b + s, 0)),
    ],
    out_specs=pl.BlockSpec((1, C, SCH), lambda b, s: (b, 0, s)),
    out_shape=jax.ShapeDtypeStruct((NB, C, S), jnp.float32),
    compiler_params=pltpu.CompilerParams(
        dimension_semantics=("parallel", "parallel")),
)


def kernel(voxel_features, coors, batch_size):
    del batch_size
    coors_flat = coors.reshape(-1)
    staging, winner = _k1(coors_flat, voxel_features)
    out = _k2(winner.reshape(TOT // SCH, 1, SCH), staging)
    return out.reshape(NB, C, NY, NX)


# speedup vs baseline: 1.0430x; 1.0002x over previous
import functools

import jax
import jax.numpy as jnp
from jax import lax
from jax.experimental import pallas as pl
from jax.experimental.pallas import tpu as pltpu
from jax.experimental.pallas import tpu_sc as plsc

NY, NX, C = 162, 162, 128
NB = 4
S = NY * NX
SP = 26624
TOT = NB * SP
P = 40000

NTILES = 32
RANGE = TOT // NTILES
CHUNK = 2512
PPAD = 16 * CHUNK
NGRP = PPAD // 16
LCAP = RANGE + 128
KCH = 3

_mesh = plsc.VectorSubcoreMesh(core_axis_name="c", subcore_axis_name="s")


@functools.partial(
    pl.kernel,
    out_type=[
        jax.ShapeDtypeStruct((TOT + 128, C), jnp.float32),
        jax.ShapeDtypeStruct((TOT,), jnp.int32),
    ],
    mesh=_mesh,
    scratch_types=[
        pltpu.VMEM((PPAD // 4,), jnp.int32),
        pltpu.VMEM((RANGE,), jnp.int32),
        pltpu.VMEM((TOT // 8,), jnp.int32),
        pltpu.VMEM((RANGE,), jnp.int32),
        pltpu.VMEM((4 * CHUNK,), jnp.int32),
        pltpu.VMEM((KCH * 128, C), jnp.float32),
        pltpu.VMEM((LCAP,), jnp.int32),
        pltpu.VMEM((LCAP,), jnp.int32),
        pltpu.VMEM((LCAP // 128, 128), jnp.int32),
        pltpu.VMEM_SHARED((PPAD,), jnp.int32),
        pltpu.VMEM_SHARED((16, TOT // 8), jnp.int32),
        pltpu.SemaphoreType.DMA,
        pltpu.SemaphoreType.DMA,
    ],
    compiler_params=pltpu.CompilerParams(needs_layout_passes=False),
)
def _k1(coors_hbm, vf_hbm, staging_hbm, winner_hbm,
        g2_v, winner_v, part_v, tmp_v, coorsw_v, rows_v, wl_v, sl_v,
        sl2_v, g2_sp, part_sp, gsem, ssem):
    sid = lax.axis_index("s")
    wid = lax.axis_index("c") * 16 + sid
    lo = wid * RANGE
    lane = jnp.arange(16, dtype=jnp.int32)
    neg1 = jnp.full((16,), -1, jnp.int32)

    cb = sid * CHUNK

    @pl.when(sid < 15)
    def _():
        pltpu.sync_copy(coors_hbm.at[pl.ds(cb * 4, 4 * CHUNK)], coorsw_v)

    @pl.when(sid == 15)
    def _():
        n_tail = 4 * (P - 15 * CHUNK)
        pltpu.sync_copy(coors_hbm.at[pl.ds(cb * 4, n_tail)],
                        coorsw_v.at[pl.ds(0, n_tail)])

    def _ga(i, _):
        off = i * 64 + lane * 4
        b = plsc.load_gather(coorsw_v, [off]) & 3
        y = plsc.load_gather(coorsw_v, [off + 2])
        x = plsc.load_gather(coorsw_v, [off + 3])
        g2 = b * SP + y * NX + x
        pmask = (cb + i * 16 + lane) < P
        g2_v[pl.ds(i * 16, 16)] = jnp.where(pmask, g2, TOT)
        return 0
    lax.fori_loop(0, CHUNK // 16, _ga, 0)
    pltpu.sync_copy(g2_v.at[pl.ds(0, CHUNK)], g2_sp.at[pl.ds(cb, CHUNK)])
    plsc.subcore_barrier()

    CL = TOT // 8
    QP = PPAD // 4
    cl = sid // 4
    qt = sid % 4
    clo = lax.axis_index("c") * (TOT // 2) + cl * CL

    def _init(i, _):
        part_v[pl.ds(i * 16, 16)] = neg1
        return 0
    lax.fori_loop(0, CL // 16, _init, 0)

    pltpu.sync_copy(g2_sp.at[pl.ds(qt * QP, QP)], g2_v)

    def _scan(i4, _):
        for u in range(4):
            i = i4 * 4 + u
            il = g2_v[pl.ds(i * 16, 16)] - clo
            m = (il >= 0) & (il < CL)
            ilc = jnp.where(m, il, 0)
            p_vec = qt * QP + i * 16 + lane
            plsc.store_scatter(part_v, [ilc], p_vec, mask=m)
        return 0
    lax.fori_loop(0, QP // 16 // 4, _scan, 0)

    pltpu.sync_copy(part_v, part_sp.at[sid])
    plsc.subcore_barrier()

    sub = qt * RANGE
    pltpu.sync_copy(part_sp.at[cl * 4, pl.ds(sub, RANGE)], winner_v)
    for q2 in range(1, 4):
        pltpu.sync_copy(part_sp.at[cl * 4 + q2, pl.ds(sub, RANGE)], tmp_v)

        def _mx(i, _):
            winner_v[pl.ds(i * 16, 16)] = jnp.maximum(
                winner_v[pl.ds(i * 16, 16)], tmp_v[pl.ds(i * 16, 16)])
            return 0
        lax.fori_loop(0, RANGE // 16, _mx, 0)

    pltpu.sync_copy(winner_v, winner_hbm.at[pl.ds(lo, RANGE)])

    def _pref(i, _):
        pos = i * 16 + lane
        wl_v[pl.ds(i * 16, 16)] = (pos + wid * 128) & 16383
        sl_v[pl.ds(i * 16, 16)] = TOT + ((pos + wid * 4) & 127)
        return 0
    lax.fori_loop(0, LCAP // 16, _pref, 0)

    def _cmp(j, acc):
        w16 = winner_v[pl.ds(j * 16, 16)]
        valid = w16 >= 0
        plsc.store_compressed(wl_v.at[pl.ds(acc, 16)], w16, mask=valid)
        slot = lo + j * 16 + lane
        plsc.store_compressed(sl_v.at[pl.ds(acc, 16)], slot, mask=valid)
        return acc + jnp.sum(valid.astype(jnp.int32))
    nv = lax.fori_loop(0, RANGE // 16, _cmp, jnp.int32(0))

    def _c2o(kk, _):
        def _c2i(j, _):
            sl2_v[kk, pl.ds(j * 16, 16)] = sl_v[pl.ds(kk * 128 + j * 16, 16)]
            return 0
        lax.fori_loop(0, 8, _c2i, 0)
        return 0
    lax.fori_loop(0, LCAP // 128, _c2o, 0)

    nsc = (nv + (KCH * 128 - 1)) // (KCH * 128)

    def _sc(t, _):
        base = t * (KCH * 128)
        hs = [pltpu.async_copy(
                  vf_hbm.at[wl_v.at[pl.ds(base + q * 128, 128)]],
                  rows_v.at[pl.ds(q * 128, 128), :], gsem)
              for q in range(KCH)]
        for h in hs:
            h.wait()
        hs2 = [pltpu.async_copy(
                   rows_v.at[pl.ds(q * 128, 128), :],
                   staging_hbm.at[sl2_v.at[t * KCH + q]], ssem)
               for q in range(KCH)]
        for h in hs2:
            h.wait()
        return 0
    lax.fori_loop(0, nsc, _sc, 0)


SCH = 13312


def _k2_body(wref, sref, oref):
    x = sref[...]
    wm = wref[0, 0]
    xt = x.T
    oref[0] = jnp.where((wm >= 0)[None, :], xt, 0.0)


_k2 = pl.pallas_call(
    _k2_body,
    grid=(NB, SP // SCH),
    in_specs=[
        pl.BlockSpec((1, 1, SCH), lambda b, s: ((SP // SCH) * b + s, 0, 0)),
        pl.BlockSpec((SCH, C), lambda b, s: ((SP // SCH) * b + s, 0)),
    ],
    out_specs=pl.BlockSpec((1, C, SCH), lambda b, s: (b, 0, s)),
    out_shape=jax.ShapeDtypeStruct((NB, C, S), jnp.float32),
    compiler_params=pltpu.CompilerParams(
        dimension_semantics=("parallel", "parallel")),
)


def kernel(voxel_features, coors, batch_size):
    del batch_size
    coors_flat = coors.reshape(-1)
    staging, winner = _k1(coors_flat, voxel_features)
    out = _k2(winner.reshape(TOT // SCH, 1, SCH), staging)
    return out.reshape(NB, C, NY, NX)
